# baseline (speedup 1.0000x reference)
import jax
import jax.numpy as jnp
from jax import lax
from jax.experimental import pallas as pl

_NUM_CLASSES = 128
_H = 224
_W = 224
_P = _H * _W
_CB = 8


def _onehot_body(x_ref, o_ref):
    c0 = pl.program_id(1) * _CB
    xv = x_ref[0]
    classes = c0 + lax.broadcasted_iota(jnp.int32, (_CB, 1), 0)
    o_ref[0] = (xv == classes).astype(jnp.float32)


def kernel(x):
    b = x.shape[0]
    x3 = x.astype(jnp.int32).reshape(b, 1, _P)
    out = pl.pallas_call(
        _onehot_body,
        grid=(b, _NUM_CLASSES // _CB),
        in_specs=[
            pl.BlockSpec((1, 1, _P), lambda i, j: (i, 0, 0)),
        ],
        out_specs=pl.BlockSpec((1, _CB, _P), lambda i, j: (i, j, 0)),
        out_shape=jax.ShapeDtypeStruct((b, _NUM_CLASSES, _P), jnp.float32),
    )(x3)
    return out.reshape(b, _NUM_CLASSES, _H, _W)


# speedup vs baseline: 1.1101x; 1.1101x over previous
import jax
import jax.numpy as jnp
from jax import lax
from jax.experimental import pallas as pl

_NUM_CLASSES = 128
_H = 224
_W = 224
_P = _H * _W
_CB = 32


def _onehot_body(x_ref, o_ref):
    c0 = pl.program_id(1) * _CB
    xv = x_ref[0]
    classes = c0 + lax.broadcasted_iota(jnp.int32, (_CB, 1), 0)
    o_ref[0] = (xv == classes).astype(jnp.float32)


def kernel(x):
    b = x.shape[0]
    x3 = x.astype(jnp.int32).reshape(b, 1, _P)
    out = pl.pallas_call(
        _onehot_body,
        grid=(b, _NUM_CLASSES // _CB),
        in_specs=[
            pl.BlockSpec((1, 1, _P), lambda i, j: (i, 0, 0)),
        ],
        out_specs=pl.BlockSpec((1, _CB, _P), lambda i, j: (i, j, 0)),
        out_shape=jax.ShapeDtypeStruct((b, _NUM_CLASSES, _P), jnp.float32),
    )(x3)
    return out.reshape(b, _NUM_CLASSES, _H, _W)
